# baseline (speedup 1.0000x reference)
import functools

import jax
import jax.numpy as jnp
import numpy as np
from jax import lax
from jax.experimental import pallas as pl
from jax.experimental.pallas import tpu as pltpu
from jax.experimental.pallas import tpu_sc as plsc

_F = 26
_E = 16
_B = 4096
_FE = _F * _E
_FIELD_DIM = 38462
_OFFS = np.arange(_F, dtype=np.int32) * _FIELD_DIM

_NC = 2
_NS = 16
_NW = _NC * _NS
_ROWS = _B * _F
_RPT = _ROWS // _NW
_CHUNK = 128
_NCH = _RPT // _CHUNK

@functools.cache
def _make_sc_gather():
    mesh = plsc.VectorSubcoreMesh(
        core_axis_name="c", subcore_axis_name="s", num_cores=_NC, num_subcores=_NS
    )

    @functools.partial(
        pl.kernel,
        mesh=mesh,
        out_type=jax.ShapeDtypeStruct((_ROWS, _E), jnp.float32),
        scratch_types=[
            pltpu.VMEM((_NCH, _CHUNK), jnp.int32),
            pltpu.VMEM((_RPT, _E), jnp.float32),
            pltpu.SemaphoreType.DMA,
        ],
        compiler_params=pltpu.CompilerParams(use_tc_tiling_on_sc=False),
    )
    def _sc_gather(idx_hbm, table_hbm, out_hbm, idx_v, rows_v, sem):
        wid = lax.axis_index("s") * _NC + lax.axis_index("c")
        pltpu.sync_copy(idx_hbm.at[wid], idx_v)
        cps = [
            pltpu.async_copy(
                table_hbm.at[idx_v.at[c]],
                rows_v.at[pl.ds(c * _CHUNK, _CHUNK)],
                sem,
            )
            for c in range(_NCH)
        ]
        for cp in cps:
            cp.wait()
        pltpu.sync_copy(rows_v, out_hbm.at[pl.ds(wid * _RPT, _RPT)])

    return _sc_gather


_GRID = 8
_BB = _B // _GRID


def _tc_fm_body(a_ref, m_ref, w_ref, b_ref, o_ref):
    a = a_ref[...]
    am = jnp.dot(a, m_ref[...], preferred_element_type=jnp.float32)
    t = jnp.sum(a * (am + w_ref[...]), axis=1)[:, None] + b_ref[...]
    o_ref[...] = 1.0 / (1.0 + jnp.exp(-t))


_tc_fm = pl.pallas_call(
    _tc_fm_body,
    grid=(_GRID,),
    in_specs=[
        pl.BlockSpec((_BB, _FE), lambda i: (i, 0)),
        pl.BlockSpec((_FE, _FE), lambda i: (0, 0)),
        pl.BlockSpec((1, _FE), lambda i: (0, 0)),
        pl.BlockSpec((1, 1), lambda i: (0, 0)),
    ],
    out_specs=pl.BlockSpec((_BB, 1), lambda i: (i, 0)),
    out_shape=jax.ShapeDtypeStruct((_B, 1), jnp.float32),
)


def kernel(x, embed_table, field_cov_w, fwfm_linear_w, bias):
    x = x.astype(jnp.int32)
    idx = (x + jnp.asarray(_OFFS)[None, :]).reshape(_NW, _NCH, _CHUNK)
    rows = _make_sc_gather()(idx, embed_table)
    a = rows.reshape(_B, _FE)

    sym = 0.5 * (field_cov_w + field_cov_w.T)
    s0 = 0.5 * sym * (1.0 - jnp.eye(_F, dtype=jnp.float32))
    m = jnp.kron(s0, jnp.eye(_E, dtype=jnp.float32))
    wflat = fwfm_linear_w.reshape(1, _FE)

    out = _tc_fm(a, m, wflat, bias.reshape(1, 1))
    return out.reshape(_B)
